# Initial kernel scaffold; baseline (speedup 1.0000x reference)
import jax
import jax.numpy as jnp
from jax import lax
from jax.experimental import pallas as pl
from jax.experimental.pallas import tpu as pltpu
from jax.experimental.pallas import tpu_sc as plsc

NC = 2
NS = 16
NW = NC * NS

BATCH = 16384
SEQ = 50
DIM = 64
B = BATCH * SEQ
BPW = B // NW
CHUNK = 512
NCHUNK = BPW // CHUNK


def _gather_body(x_hbm, table_hbm, out_hbm, idx_v, rows_v, sem):
    wid = lax.axis_index("s") * NC + lax.axis_index("c")
    base = wid * BPW

    def chunk(i, carry):
        off = base + i * CHUNK
        pltpu.sync_copy(x_hbm.at[pl.ds(off, CHUNK)], idx_v)
        pltpu.async_copy(table_hbm.at[idx_v], rows_v, sem).wait()
        pltpu.sync_copy(rows_v, out_hbm.at[pl.ds(off, CHUNK)])
        return carry

    lax.fori_loop(0, NCHUNK, chunk, 0)


@jax.jit
def _embedding_lookup(x_flat, table):
    mesh = plsc.VectorSubcoreMesh(core_axis_name="c", subcore_axis_name="s")
    return pl.kernel(
        _gather_body,
        out_type=jax.ShapeDtypeStruct((B, DIM), jnp.float32),
        mesh=mesh,
        scratch_types=[
            pltpu.VMEM((CHUNK,), jnp.int32),
            pltpu.VMEM((CHUNK, DIM), jnp.float32),
            pltpu.SemaphoreType.DMA,
        ],
    )(x_flat, table)


def kernel(x, table):
    out = _embedding_lookup(x.reshape(-1).astype(jnp.int32), table)
    return out.reshape(BATCH, SEQ, DIM)


# speedup vs baseline: 1.7962x; 1.7962x over previous
import jax
import jax.numpy as jnp
from jax import lax
from jax.experimental import pallas as pl
from jax.experimental.pallas import tpu as pltpu
from jax.experimental.pallas import tpu_sc as plsc

NC = 2
NS = 16
NW = NC * NS

BATCH = 16384
SEQ = 50
DIM = 64
B = BATCH * SEQ
BPW = B // NW
CHUNK = 512
NCHUNK = BPW // CHUNK


def _gather_body(x_hbm, table_hbm, out_hbm, idx_v, rows_v, sem):
    wid = lax.axis_index("s") * NC + lax.axis_index("c")
    base = wid * BPW

    def chunk(i, carry):
        off = base + i * CHUNK
        pltpu.sync_copy(x_hbm.at[pl.ds(off, CHUNK)], idx_v)
        pltpu.async_copy(table_hbm.at[idx_v], rows_v, sem).wait()
        pltpu.sync_copy(rows_v, out_hbm.at[pl.ds(off, CHUNK)])
        return carry

    lax.fori_loop(0, NCHUNK, chunk, 0)


@jax.jit
def _embedding_lookup(x_flat, table):
    mesh = plsc.VectorSubcoreMesh(core_axis_name="c", subcore_axis_name="s")
    return pl.kernel(
        _gather_body,
        out_type=jax.ShapeDtypeStruct((B, DIM), jnp.float32),
        mesh=mesh,
        scratch_types=[
            pltpu.VMEM((CHUNK,), jnp.int32),
            pltpu.VMEM((CHUNK, DIM), jnp.float32),
            pltpu.SemaphoreType.DMA,
        ],
        compiler_params=pltpu.CompilerParams(use_tc_tiling_on_sc=False),
    )(x_flat, table)


def kernel(x, table):
    out = _embedding_lookup(x.reshape(-1).astype(jnp.int32), table)
    return out.reshape(BATCH, SEQ, DIM)


# speedup vs baseline: 1.8749x; 1.0438x over previous
import jax
import jax.numpy as jnp
from jax import lax
from jax.experimental import pallas as pl
from jax.experimental.pallas import tpu as pltpu
from jax.experimental.pallas import tpu_sc as plsc

NC = 2
NS = 16
NW = NC * NS

BATCH = 16384
SEQ = 50
DIM = 64
B = BATCH * SEQ
BPW = B // NW
CHUNK = 512
NCHUNK = BPW // CHUNK
NBUF = 2
assert NCHUNK % NBUF == 0


def _gather_body(x_hbm, table_hbm, out_hbm, idx_all, rows, gsem, ssem, isem):
    wid = lax.axis_index("s") * NC + lax.axis_index("c")
    base = wid * BPW

    pltpu.async_copy(x_hbm.at[pl.ds(base, BPW)], idx_all, isem).wait()

    def gather_start(j, bj):
        src = table_hbm.at[idx_all.at[pl.ds(j * CHUNK, CHUNK)]]
        pltpu.async_copy(src, rows.at[bj], gsem.at[bj])

    def gather_wait(b):
        pltpu.make_async_copy(table_hbm.at[idx_all.at[pl.ds(0, CHUNK)]],
                              rows.at[b], gsem.at[b]).wait()

    def store_start(i, b):
        pltpu.async_copy(rows.at[b], out_hbm.at[pl.ds(base + i * CHUNK, CHUNK)],
                         ssem.at[b])

    def store_wait(b):
        pltpu.make_async_copy(rows.at[b], out_hbm.at[pl.ds(base, CHUNK)],
                              ssem.at[b]).wait()

    for b in range(NBUF - 1):
        gather_start(b, b)

    def group(g, carry):
        for b in range(NBUF):
            i = g * NBUF + b
            j = i + NBUF - 1
            bj = (b + NBUF - 1) % NBUF

            @pl.when(j < NCHUNK)
            def _():
                @pl.when(j >= NBUF)
                def _():
                    store_wait(bj)
                gather_start(j, bj)

            gather_wait(b)
            store_start(i, b)
        return carry

    lax.fori_loop(0, NCHUNK // NBUF, group, 0)

    for b in range(NBUF):
        store_wait(b)


@jax.jit
def _embedding_lookup(x_flat, table):
    mesh = plsc.VectorSubcoreMesh(core_axis_name="c", subcore_axis_name="s")
    return pl.kernel(
        _gather_body,
        out_type=jax.ShapeDtypeStruct((B, DIM), jnp.float32),
        mesh=mesh,
        scratch_types=[
            pltpu.VMEM((BPW,), jnp.int32),
            pltpu.VMEM((NBUF, CHUNK, DIM), jnp.float32),
            pltpu.SemaphoreType.DMA((NBUF,)),
            pltpu.SemaphoreType.DMA((NBUF,)),
            pltpu.SemaphoreType.DMA,
        ],
        compiler_params=pltpu.CompilerParams(use_tc_tiling_on_sc=False),
    )(x_flat, table)


def kernel(x, table):
    out = _embedding_lookup(x.reshape(-1).astype(jnp.int32), table)
    return out.reshape(BATCH, SEQ, DIM)
